# baseline (speedup 1.0000x reference)
import functools

import jax
import jax.numpy as jnp
from jax import lax
from jax.experimental import pallas as pl
from jax.experimental.pallas import tpu as pltpu
from jax.experimental.pallas import tpu_sc as plsc

_L = 16
_EPS = 1e-8


def _rsqrt(x):
    i = lax.bitcast_convert_type(x, jnp.int32)
    i = jnp.int32(0x5F3759DF) - lax.shift_right_arithmetic(i, 1)
    y = lax.bitcast_convert_type(i, jnp.float32)
    for _ in range(3):
        y = y * (1.5 - 0.5 * x * y * y)
    return y


@functools.lru_cache(maxsize=None)
def _make_sc_kernel(N, D, E):
    info = plsc.get_sparse_core_info()
    NC, NS = info.num_cores, info.num_subcores
    NW = NC * NS
    assert E % NW == 0 and D % _L == 0
    EPW = E // NW
    C = 128
    n_full = EPW // C
    tail = EPW - n_full * C
    assert tail % _L == 0 and n_full % 2 == 0

    mesh = plsc.VectorSubcoreMesh(core_axis_name="c", subcore_axis_name="s")

    @functools.partial(
        pl.kernel,
        out_type=jax.ShapeDtypeStruct((E,), jnp.float32),
        mesh=mesh,
        compiler_params=pltpu.CompilerParams(needs_layout_passes=False),
        scratch_types=[
            pltpu.VMEM((EPW,), jnp.int32),
            pltpu.VMEM((EPW,), jnp.int32),
            pltpu.VMEM((EPW,), jnp.float32),
            pltpu.VMEM((C, D), jnp.float32),
            pltpu.VMEM((C, D), jnp.float32),
            pltpu.VMEM((C, D), jnp.float32),
            pltpu.VMEM((C, D), jnp.float32),
            pltpu.VMEM((C,), jnp.float32),
            pltpu.VMEM((C,), jnp.float32),
            pltpu.VMEM((C,), jnp.float32),
            pltpu.SemaphoreType.DMA,
            pltpu.SemaphoreType.DMA,
            pltpu.SemaphoreType.DMA,
            pltpu.SemaphoreType.DMA,
        ],
    )
    def cosine_sc(z_hbm, src_hbm, dst_hbm, out_hbm,
                  src_v, dst_v, out_v, a0, b0, a1, b1,
                  dbuf, nabuf, nbbuf, sa0, sb0, sa1, sb1):
        wid = lax.axis_index("s") * NC + lax.axis_index("c")
        base = wid * EPW
        pltpu.sync_copy(src_hbm.at[pl.ds(base, EPW)], src_v)
        pltpu.sync_copy(dst_hbm.at[pl.ds(base, EPW)], dst_v)

        def gather(off, size, av, bv, sa, sb):
            h = size // 2
            pltpu.async_copy(z_hbm.at[src_v.at[pl.ds(off, h)]],
                             av.at[pl.ds(0, h)], sa)
            pltpu.async_copy(z_hbm.at[src_v.at[pl.ds(off + h, h)]],
                             av.at[pl.ds(h, h)], sa)
            pltpu.async_copy(z_hbm.at[dst_v.at[pl.ds(off, h)]],
                             bv.at[pl.ds(0, h)], sb)
            pltpu.async_copy(z_hbm.at[dst_v.at[pl.ds(off + h, h)]],
                             bv.at[pl.ds(h, h)], sb)

        def wait(off, size, av, bv, sa, sb):
            h = size // 2
            pltpu.make_async_copy(z_hbm.at[src_v.at[pl.ds(off, h)]],
                                  av.at[pl.ds(0, h)], sa).wait()
            pltpu.make_async_copy(z_hbm.at[src_v.at[pl.ds(off + h, h)]],
                                  av.at[pl.ds(h, h)], sa).wait()
            pltpu.make_async_copy(z_hbm.at[dst_v.at[pl.ds(off, h)]],
                                  bv.at[pl.ds(0, h)], sb).wait()
            pltpu.make_async_copy(z_hbm.at[dst_v.at[pl.ds(off + h, h)]],
                                  bv.at[pl.ds(h, h)], sb).wait()

        def _tree8(x):
            return (((x[0] + x[1]) + (x[2] + x[3]))
                    + ((x[4] + x[5]) + (x[6] + x[7])))

        lane = lax.broadcasted_iota(jnp.int32, (_L,), 0)
        last_lane = lane == (_L - 1)

        def compute(off, size, av, bv):
            @plsc.parallel_loop(0, size, 1, unroll=4)
            def edge_body(e, av=av, bv=bv):
                a = [av[e, pl.ds(k * _L, _L)] for k in range(D // _L)]
                b = [bv[e, pl.ds(k * _L, _L)] for k in range(D // _L)]
                dot = plsc.cumsum(_tree8([a[k] * b[k]
                                          for k in range(D // _L)]))
                na2 = plsc.cumsum(_tree8([a[k] * a[k]
                                          for k in range(D // _L)]))
                nb2 = plsc.cumsum(_tree8([b[k] * b[k]
                                          for k in range(D // _L)]))
                eidx = jnp.full((_L,), e, jnp.int32)
                plsc.store_scatter(dbuf, [eidx], dot, mask=last_lane)
                plsc.store_scatter(nabuf, [eidx], na2, mask=last_lane)
                plsc.store_scatter(nbbuf, [eidx], nb2, mask=last_lane)

            for g in range(size // _L):
                dot = dbuf[pl.ds(g * _L, _L)]
                s2 = nabuf[pl.ds(g * _L, _L)] * nbbuf[pl.ds(g * _L, _L)]
                val = jnp.where(s2 >= _EPS * _EPS,
                                dot * _rsqrt(s2), dot * (1.0 / _EPS))
                out_v[pl.ds(off + g * _L, _L)] = 1.0 / (1.0 + jnp.exp(-val))

        gather(0, C, a0, b0, sa0, sb0)

        def pair_body(i, carry):
            c0 = (2 * i) * C
            c1 = (2 * i + 1) * C
            gather(c1, C, a1, b1, sa1, sb1)
            wait(c0, C, a0, b0, sa0, sb0)
            compute(c0, C, a0, b0)

            @pl.when(2 * i + 2 < n_full)
            def _():
                gather(c1 + C, C, a0, b0, sa0, sb0)

            wait(c1, C, a1, b1, sa1, sb1)
            compute(c1, C, a1, b1)
            return carry

        lax.fori_loop(0, n_full // 2, pair_body, 0)
        if tail:
            off = n_full * C
            gather(off, tail, a0, b0, sa0, sb0)
            wait(off, tail, a0, b0, sa0, sb0)
            compute(off, tail, a0, b0)
        pltpu.sync_copy(out_v, out_hbm.at[pl.ds(base, EPW)])

    return cosine_sc


def kernel(z, edge_index):
    N, D = z.shape
    E = edge_index.shape[1]
    src = edge_index[0].astype(jnp.int32)
    dst = edge_index[1].astype(jnp.int32)
    return _make_sc_kernel(N, D, E)(z, src, dst)


# speedup vs baseline: 1.0171x; 1.0171x over previous
import functools

import jax
import jax.numpy as jnp
from jax import lax
from jax.experimental import pallas as pl
from jax.experimental.pallas import tpu as pltpu
from jax.experimental.pallas import tpu_sc as plsc

_L = 16
_EPS = 1e-8


def _rsqrt(x):
    i = lax.bitcast_convert_type(x, jnp.int32)
    i = jnp.int32(0x5F3759DF) - lax.shift_right_arithmetic(i, 1)
    y = lax.bitcast_convert_type(i, jnp.float32)
    for _ in range(3):
        y = y * (1.5 - 0.5 * x * y * y)
    return y


@functools.lru_cache(maxsize=None)
def _make_sc_kernel(N, D, E):
    info = plsc.get_sparse_core_info()
    NC, NS = info.num_cores, info.num_subcores
    NW = NC * NS
    assert E % NW == 0 and D % _L == 0
    EPW = E // NW
    C = 160
    n_full = EPW // C
    tail = EPW - n_full * C
    assert tail % _L == 0 and n_full % 2 == 0

    mesh = plsc.VectorSubcoreMesh(core_axis_name="c", subcore_axis_name="s")

    @functools.partial(
        pl.kernel,
        out_type=jax.ShapeDtypeStruct((E,), jnp.float32),
        mesh=mesh,
        compiler_params=pltpu.CompilerParams(needs_layout_passes=False),
        scratch_types=[
            pltpu.VMEM((EPW,), jnp.int32),
            pltpu.VMEM((EPW,), jnp.int32),
            pltpu.VMEM((EPW,), jnp.float32),
            pltpu.VMEM((C, D), jnp.float32),
            pltpu.VMEM((C, D), jnp.float32),
            pltpu.VMEM((C, D), jnp.float32),
            pltpu.VMEM((C, D), jnp.float32),
            pltpu.VMEM((C,), jnp.float32),
            pltpu.VMEM((C,), jnp.float32),
            pltpu.VMEM((C,), jnp.float32),
            pltpu.SemaphoreType.DMA,
            pltpu.SemaphoreType.DMA,
            pltpu.SemaphoreType.DMA,
            pltpu.SemaphoreType.DMA,
        ],
    )
    def cosine_sc(z_hbm, src_hbm, dst_hbm, out_hbm,
                  src_v, dst_v, out_v, a0, b0, a1, b1,
                  dbuf, nabuf, nbbuf, sa0, sb0, sa1, sb1):
        wid = lax.axis_index("s") * NC + lax.axis_index("c")
        base = wid * EPW
        pltpu.sync_copy(src_hbm.at[pl.ds(base, EPW)], src_v)
        pltpu.sync_copy(dst_hbm.at[pl.ds(base, EPW)], dst_v)

        def gather(off, size, av, bv, sa, sb):
            h = size // 2
            pltpu.async_copy(z_hbm.at[src_v.at[pl.ds(off, h)]],
                             av.at[pl.ds(0, h)], sa)
            pltpu.async_copy(z_hbm.at[src_v.at[pl.ds(off + h, h)]],
                             av.at[pl.ds(h, h)], sa)
            pltpu.async_copy(z_hbm.at[dst_v.at[pl.ds(off, h)]],
                             bv.at[pl.ds(0, h)], sb)
            pltpu.async_copy(z_hbm.at[dst_v.at[pl.ds(off + h, h)]],
                             bv.at[pl.ds(h, h)], sb)

        def wait(off, size, av, bv, sa, sb):
            h = size // 2
            pltpu.make_async_copy(z_hbm.at[src_v.at[pl.ds(off, h)]],
                                  av.at[pl.ds(0, h)], sa).wait()
            pltpu.make_async_copy(z_hbm.at[src_v.at[pl.ds(off + h, h)]],
                                  av.at[pl.ds(h, h)], sa).wait()
            pltpu.make_async_copy(z_hbm.at[dst_v.at[pl.ds(off, h)]],
                                  bv.at[pl.ds(0, h)], sb).wait()
            pltpu.make_async_copy(z_hbm.at[dst_v.at[pl.ds(off + h, h)]],
                                  bv.at[pl.ds(h, h)], sb).wait()

        def _tree8(x):
            return (((x[0] + x[1]) + (x[2] + x[3]))
                    + ((x[4] + x[5]) + (x[6] + x[7])))

        lane = lax.broadcasted_iota(jnp.int32, (_L,), 0)
        last_lane = lane == (_L - 1)

        def compute(off, size, av, bv):
            @plsc.parallel_loop(0, size, 1, unroll=4)
            def edge_body(e, av=av, bv=bv):
                a = [av[e, pl.ds(k * _L, _L)] for k in range(D // _L)]
                b = [bv[e, pl.ds(k * _L, _L)] for k in range(D // _L)]
                dot = plsc.cumsum(_tree8([a[k] * b[k]
                                          for k in range(D // _L)]))
                na2 = plsc.cumsum(_tree8([a[k] * a[k]
                                          for k in range(D // _L)]))
                nb2 = plsc.cumsum(_tree8([b[k] * b[k]
                                          for k in range(D // _L)]))
                eidx = jnp.full((_L,), e, jnp.int32)
                plsc.store_scatter(dbuf, [eidx], dot, mask=last_lane)
                plsc.store_scatter(nabuf, [eidx], na2, mask=last_lane)
                plsc.store_scatter(nbbuf, [eidx], nb2, mask=last_lane)

            for g in range(size // _L):
                dot = dbuf[pl.ds(g * _L, _L)]
                s2 = nabuf[pl.ds(g * _L, _L)] * nbbuf[pl.ds(g * _L, _L)]
                val = jnp.where(s2 >= _EPS * _EPS,
                                dot * _rsqrt(s2), dot * (1.0 / _EPS))
                out_v[pl.ds(off + g * _L, _L)] = 1.0 / (1.0 + jnp.exp(-val))

        gather(0, C, a0, b0, sa0, sb0)

        def pair_body(i, carry):
            c0 = (2 * i) * C
            c1 = (2 * i + 1) * C
            gather(c1, C, a1, b1, sa1, sb1)
            wait(c0, C, a0, b0, sa0, sb0)
            compute(c0, C, a0, b0)

            @pl.when(2 * i + 2 < n_full)
            def _():
                gather(c1 + C, C, a0, b0, sa0, sb0)

            wait(c1, C, a1, b1, sa1, sb1)
            compute(c1, C, a1, b1)
            return carry

        lax.fori_loop(0, n_full // 2, pair_body, 0)
        if tail:
            off = n_full * C
            gather(off, tail, a0, b0, sa0, sb0)
            wait(off, tail, a0, b0, sa0, sb0)
            compute(off, tail, a0, b0)
        pltpu.sync_copy(out_v, out_hbm.at[pl.ds(base, EPW)])

    return cosine_sc


def kernel(z, edge_index):
    N, D = z.shape
    E = edge_index.shape[1]
    src = edge_index[0].astype(jnp.int32)
    dst = edge_index[1].astype(jnp.int32)
    return _make_sc_kernel(N, D, E)(z, src, dst)


# speedup vs baseline: 1.0294x; 1.0121x over previous
import functools

import jax
import jax.numpy as jnp
from jax import lax
from jax.experimental import pallas as pl
from jax.experimental.pallas import tpu as pltpu
from jax.experimental.pallas import tpu_sc as plsc

_L = 16
_EPS = 1e-8


def _rsqrt(x):
    i = lax.bitcast_convert_type(x, jnp.int32)
    i = jnp.int32(0x5F3759DF) - lax.shift_right_arithmetic(i, 1)
    y = lax.bitcast_convert_type(i, jnp.float32)
    for _ in range(3):
        y = y * (1.5 - 0.5 * x * y * y)
    return y


@functools.lru_cache(maxsize=None)
def _make_sc_kernel(N, D, E):
    info = plsc.get_sparse_core_info()
    NC, NS = info.num_cores, info.num_subcores
    NW = NC * NS
    assert E % NW == 0 and D % _L == 0
    EPW = E // NW
    C = 192
    n_full = EPW // C
    tail = EPW - n_full * C
    assert tail % _L == 0 and n_full % 2 == 0

    mesh = plsc.VectorSubcoreMesh(core_axis_name="c", subcore_axis_name="s")

    @functools.partial(
        pl.kernel,
        out_type=jax.ShapeDtypeStruct((E,), jnp.float32),
        mesh=mesh,
        compiler_params=pltpu.CompilerParams(needs_layout_passes=False),
        scratch_types=[
            pltpu.VMEM((EPW,), jnp.int32),
            pltpu.VMEM((EPW,), jnp.int32),
            pltpu.VMEM((EPW,), jnp.float32),
            pltpu.VMEM((C, D), jnp.float32),
            pltpu.VMEM((C, D), jnp.float32),
            pltpu.VMEM((C, D), jnp.float32),
            pltpu.VMEM((C, D), jnp.float32),
            pltpu.VMEM((C,), jnp.float32),
            pltpu.VMEM((C,), jnp.float32),
            pltpu.VMEM((C,), jnp.float32),
            pltpu.SemaphoreType.DMA,
            pltpu.SemaphoreType.DMA,
            pltpu.SemaphoreType.DMA,
            pltpu.SemaphoreType.DMA,
        ],
    )
    def cosine_sc(z_hbm, src_hbm, dst_hbm, out_hbm,
                  src_v, dst_v, out_v, a0, b0, a1, b1,
                  dbuf, nabuf, nbbuf, sa0, sb0, sa1, sb1):
        wid = lax.axis_index("s") * NC + lax.axis_index("c")
        base = wid * EPW
        pltpu.sync_copy(src_hbm.at[pl.ds(base, EPW)], src_v)
        pltpu.sync_copy(dst_hbm.at[pl.ds(base, EPW)], dst_v)

        def gather(off, size, av, bv, sa, sb):
            h = size // 2
            pltpu.async_copy(z_hbm.at[src_v.at[pl.ds(off, h)]],
                             av.at[pl.ds(0, h)], sa)
            pltpu.async_copy(z_hbm.at[src_v.at[pl.ds(off + h, h)]],
                             av.at[pl.ds(h, h)], sa)
            pltpu.async_copy(z_hbm.at[dst_v.at[pl.ds(off, h)]],
                             bv.at[pl.ds(0, h)], sb)
            pltpu.async_copy(z_hbm.at[dst_v.at[pl.ds(off + h, h)]],
                             bv.at[pl.ds(h, h)], sb)

        def wait(off, size, av, bv, sa, sb):
            h = size // 2
            pltpu.make_async_copy(z_hbm.at[src_v.at[pl.ds(off, h)]],
                                  av.at[pl.ds(0, h)], sa).wait()
            pltpu.make_async_copy(z_hbm.at[src_v.at[pl.ds(off + h, h)]],
                                  av.at[pl.ds(h, h)], sa).wait()
            pltpu.make_async_copy(z_hbm.at[dst_v.at[pl.ds(off, h)]],
                                  bv.at[pl.ds(0, h)], sb).wait()
            pltpu.make_async_copy(z_hbm.at[dst_v.at[pl.ds(off + h, h)]],
                                  bv.at[pl.ds(h, h)], sb).wait()

        def _tree8(x):
            return (((x[0] + x[1]) + (x[2] + x[3]))
                    + ((x[4] + x[5]) + (x[6] + x[7])))

        lane = lax.broadcasted_iota(jnp.int32, (_L,), 0)
        last_lane = lane == (_L - 1)

        def compute(off, size, av, bv):
            @plsc.parallel_loop(0, size, 1, unroll=4)
            def edge_body(e, av=av, bv=bv):
                a = [av[e, pl.ds(k * _L, _L)] for k in range(D // _L)]
                b = [bv[e, pl.ds(k * _L, _L)] for k in range(D // _L)]
                dot = plsc.cumsum(_tree8([a[k] * b[k]
                                          for k in range(D // _L)]))
                na2 = plsc.cumsum(_tree8([a[k] * a[k]
                                          for k in range(D // _L)]))
                nb2 = plsc.cumsum(_tree8([b[k] * b[k]
                                          for k in range(D // _L)]))
                eidx = jnp.full((_L,), e, jnp.int32)
                plsc.store_scatter(dbuf, [eidx], dot, mask=last_lane)
                plsc.store_scatter(nabuf, [eidx], na2, mask=last_lane)
                plsc.store_scatter(nbbuf, [eidx], nb2, mask=last_lane)

            for g in range(size // _L):
                dot = dbuf[pl.ds(g * _L, _L)]
                s2 = nabuf[pl.ds(g * _L, _L)] * nbbuf[pl.ds(g * _L, _L)]
                val = jnp.where(s2 >= _EPS * _EPS,
                                dot * _rsqrt(s2), dot * (1.0 / _EPS))
                out_v[pl.ds(off + g * _L, _L)] = 1.0 / (1.0 + jnp.exp(-val))

        gather(0, C, a0, b0, sa0, sb0)

        def pair_body(i, carry):
            c0 = (2 * i) * C
            c1 = (2 * i + 1) * C
            gather(c1, C, a1, b1, sa1, sb1)
            wait(c0, C, a0, b0, sa0, sb0)
            compute(c0, C, a0, b0)

            @pl.when(2 * i + 2 < n_full)
            def _():
                gather(c1 + C, C, a0, b0, sa0, sb0)

            wait(c1, C, a1, b1, sa1, sb1)
            compute(c1, C, a1, b1)
            return carry

        lax.fori_loop(0, n_full // 2, pair_body, 0)
        if tail:
            off = n_full * C
            gather(off, tail, a0, b0, sa0, sb0)
            wait(off, tail, a0, b0, sa0, sb0)
            compute(off, tail, a0, b0)
        pltpu.sync_copy(out_v, out_hbm.at[pl.ds(base, EPW)])

    return cosine_sc


def kernel(z, edge_index):
    N, D = z.shape
    E = edge_index.shape[1]
    src = edge_index[0].astype(jnp.int32)
    dst = edge_index[1].astype(jnp.int32)
    return _make_sc_kernel(N, D, E)(z, src, dst)
